# baseline (speedup 1.0000x reference)
import functools

import jax
import jax.numpy as jnp
from jax import lax
from jax.experimental import pallas as pl
from jax.experimental.pallas import tpu as pltpu
from jax.experimental.pallas import tpu_sc as plsc

B_, S_, D_ = 1024, 200, 4
EMBED = 64
VOCAB = 1000000
N = B_ * S_
NW = 32
ROWS_W = N // NW
C = 128
G = C * D_
SUB = 128
NSUB = G // SUB
SUPER = 2
NSUPER = ROWS_W // (C * SUPER)
LANES = 16
QE = EMBED // LANES

_mesh = plsc.VectorSubcoreMesh(core_axis_name="c", subcore_axis_name="s")

_BCAST_DNUMS = lax.GatherDimensionNumbers(
    offset_dims=(), collapsed_slice_dims=(0,), start_index_map=(0,))


def _bcast(vec, lane):
    idx = jnp.full((LANES, 1), lane, jnp.int32)
    return lax.gather(vec, idx, _BCAST_DNUMS, (1,),
                      mode=lax.GatherScatterMode.PROMISE_IN_BOUNDS)


@functools.partial(
    pl.kernel,
    out_type=jax.ShapeDtypeStruct((N // 2, 2 * EMBED), jnp.float32),
    mesh=_mesh,
    compiler_params=pltpu.CompilerParams(needs_layout_passes=False),
    scratch_types=[
        pltpu.VMEM((SUPER * NSUB, SUB), jnp.int32),
        pltpu.VMEM((SUPER * NSUB, SUB), jnp.int32),
        pltpu.VMEM((G, 2 * EMBED), jnp.float32),
        pltpu.VMEM((C // 2, 2 * EMBED), jnp.float32),
        pltpu.SemaphoreType.DMA,
    ],
)
def _emb_pool(pidx_hbm, psel_hbm, table2_hbm, out_hbm, pidx_v, psel_v, pbuf,
              obuf, sem):
    wid = lax.axis_index("s") * 2 + lax.axis_index("c")
    base = wid * ROWS_W

    def super_chunk(go, carry):
        rbase = base + go * (C * SUPER)
        irow = pl.multiple_of(rbase * D_ // SUB, SUPER * NSUB)
        pltpu.sync_copy(pidx_hbm.at[pl.ds(irow, SUPER * NSUB)], pidx_v)
        pltpu.sync_copy(psel_hbm.at[pl.ds(irow, SUPER * NSUB)], psel_v)

        for c in range(SUPER):
            descs = [
                pltpu.async_copy(
                    table2_hbm.at[pidx_v.at[c * NSUB + j]],
                    pbuf.at[pl.ds(j * SUB, SUB)],
                    sem,
                )
                for j in range(NSUB)
            ]
            for d in descs:
                d.wait()

            def pool4(j, carry2):
                p16 = psel_v[c * NSUB + (j >> 3), pl.ds((j & 7) * LANES,
                                                        LANES)]
                r0 = j * 4 * D_
                for jj in range(4):
                    accs = [None] * QE
                    for d in range(D_):
                        r = r0 + jj * D_ + d
                        m = _bcast(p16, jj * D_ + d) != 0
                        for q in range(QE):
                            lo = pbuf[r, pl.ds(q * LANES, LANES)]
                            hi = pbuf[r, pl.ds(EMBED + q * LANES, LANES)]
                            v = jnp.where(m, hi, lo)
                            accs[q] = v if accs[q] is None else accs[q] + v
                    orow = 2 * j + (jj >> 1)
                    cbase = (jj & 1) * EMBED
                    for q in range(QE):
                        obuf[orow, pl.ds(cbase + q * LANES, LANES)] = accs[q]
                return carry2

            lax.fori_loop(0, C // 4, pool4, 0)
            obase = pl.multiple_of((rbase + c * C) // 2, EMBED)
            pltpu.sync_copy(obuf, out_hbm.at[pl.ds(obase, C // 2)])
        return carry

    lax.fori_loop(0, NSUPER, super_chunk, 0)


def kernel(input, table):
    b, s, d = input.shape
    flat = input.reshape(-1)
    pidx = (flat >> 1).reshape(-1, SUB)
    psel = (flat & 1).reshape(-1, SUB)
    table2 = table.reshape(VOCAB // 2, 2 * EMBED)
    out = _emb_pool(pidx, psel, table2)
    return out.reshape(b, s, EMBED)


# speedup vs baseline: 1.2749x; 1.2749x over previous
import functools

import jax
import jax.numpy as jnp
from jax import lax
from jax.experimental import pallas as pl
from jax.experimental.pallas import tpu as pltpu
from jax.experimental.pallas import tpu_sc as plsc

B_, S_, D_ = 1024, 200, 4
EMBED = 64
N = B_ * S_
NW = 32
ROWS_W = N // NW
C = 128
G = C * D_
NCHUNK = ROWS_W // C
SUB = 128
NSUB = G // SUB
NSUPER = NCHUNK // 2
LANES = 16
QE = EMBED // LANES

_mesh = plsc.VectorSubcoreMesh(core_axis_name="c", subcore_axis_name="s")


@functools.partial(
    pl.kernel,
    out_type=jax.ShapeDtypeStruct((N // 2, 2 * EMBED), jnp.float32),
    mesh=_mesh,
    compiler_params=pltpu.CompilerParams(use_tc_tiling_on_sc=False),
    scratch_types=[
        pltpu.VMEM((2 * NSUB, SUB), jnp.int32),
        pltpu.VMEM((G, EMBED), jnp.float32),
        pltpu.VMEM((G, EMBED), jnp.float32),
        pltpu.VMEM((C // 2, 2 * EMBED), jnp.float32),
        pltpu.VMEM((C // 2, 2 * EMBED), jnp.float32),
        pltpu.SemaphoreType.DMA,
        pltpu.SemaphoreType.DMA,
    ],
)
def _emb_pool(idx_hbm, table_hbm, out_hbm, idx_v, gbuf0, gbuf1, obuf0, obuf1,
              gsem, osem):
    wid = lax.axis_index("s") * 2 + lax.axis_index("c")
    base = wid * ROWS_W
    gbufs = (gbuf0, gbuf1)
    obufs = (obuf0, obuf1)

    def stage(s):
        irow = pl.multiple_of(base * D_ // SUB + s * 2 * NSUB, 2 * NSUB)
        pltpu.sync_copy(idx_hbm.at[pl.ds(irow, 2 * NSUB)], idx_v)

    def fire(g, p):
        for j in range(NSUB):
            pltpu.async_copy(
                table_hbm.at[idx_v.at[(g % 2) * NSUB + j]],
                gbufs[p].at[pl.ds(j * SUB, SUB)],
                gsem,
            )

    def wait_gathers(p):
        for j in range(NSUB):
            pltpu.make_async_copy(
                table_hbm.at[idx_v.at[j]],
                gbufs[p].at[pl.ds(j * SUB, SUB)],
                gsem,
            ).wait()

    def obase(g):
        return pl.multiple_of(base // 2 + g * (C // 2), EMBED)

    def wait_out(p):
        pltpu.make_async_copy(
            obufs[p], out_hbm.at[pl.ds(0, C // 2)], osem).wait()

    def pool(p):
        gbuf, obuf = gbufs[p], obufs[p]

        def row(n, carry):
            r = n * D_
            orow = n >> 1
            cbase = (n & 1) * EMBED
            for q in range(QE):
                sl = pl.ds(q * LANES, LANES)
                acc = gbuf[r, sl]
                for k in range(1, D_):
                    acc = acc + gbuf[r + k, sl]
                obuf[orow, pl.ds(cbase + q * LANES, LANES)] = acc
            return carry

        lax.fori_loop(0, C, row, 0, unroll=2)

    def out_copy(g, p):
        pltpu.async_copy(obufs[p], out_hbm.at[pl.ds(obase(g), C // 2)], osem)

    stage(0)
    fire(0, 0)

    def super_body(go, carry):
        g = 2 * go
        fire(g + 1, 1)
        wait_gathers(0)

        @pl.when(go >= 1)
        def _():
            wait_out(0)

        pool(0)
        out_copy(g, 0)

        wait_gathers(1)
        stage(go + 1)
        fire(g + 2, 0)

        @pl.when(go >= 1)
        def _():
            wait_out(1)

        pool(1)
        out_copy(g + 1, 1)
        return carry

    lax.fori_loop(0, NSUPER - 1, super_body, 0)

    fire(NCHUNK - 1, 1)
    wait_gathers(0)
    wait_out(0)
    pool(0)
    out_copy(NCHUNK - 2, 0)
    wait_gathers(1)
    wait_out(1)
    pool(1)
    out_copy(NCHUNK - 1, 1)
    wait_out(0)
    wait_out(1)


def kernel(input, table):
    b, s, d = input.shape
    flat_idx = input.reshape(b * s * d // SUB, SUB)
    out = _emb_pool(flat_idx, table)
    return out.reshape(b, s, EMBED)


# speedup vs baseline: 1.3796x; 1.0821x over previous
import functools

import jax
import jax.numpy as jnp
from jax import lax
from jax.experimental import pallas as pl
from jax.experimental.pallas import tpu as pltpu
from jax.experimental.pallas import tpu_sc as plsc

B_, S_, D_ = 1024, 200, 4
EMBED = 64
N = B_ * S_
NW = 32
ROWS_W = N // NW
C = 128
G = C * D_
NCHUNK = ROWS_W // C
SUB = 128
NSUB = G // SUB
NSUPER = NCHUNK // 2
LANES = 16
QE = EMBED // LANES

_mesh = plsc.VectorSubcoreMesh(core_axis_name="c", subcore_axis_name="s")


@functools.partial(
    pl.kernel,
    out_type=jax.ShapeDtypeStruct((N, 2 * EMBED), jnp.float32),
    mesh=_mesh,
    compiler_params=pltpu.CompilerParams(use_tc_tiling_on_sc=False),
    scratch_types=[
        pltpu.VMEM((2 * NSUB, SUB), jnp.int32),
        pltpu.VMEM((G, EMBED), jnp.float32),
        pltpu.VMEM((G, EMBED), jnp.float32),
        pltpu.VMEM((C, 2 * EMBED), jnp.float32),
        pltpu.VMEM((C, 2 * EMBED), jnp.float32),
        pltpu.SemaphoreType.DMA,
        pltpu.SemaphoreType.DMA,
    ],
)
def _emb_pool(idx_hbm, table_hbm, out_hbm, idx_v, gbuf0, gbuf1, obuf0, obuf1,
              gsem, osem):
    wid = lax.axis_index("s") * 2 + lax.axis_index("c")
    base = wid * ROWS_W
    gbufs = (gbuf0, gbuf1)
    obufs = (obuf0, obuf1)

    def stage(s):
        irow = pl.multiple_of(base * D_ // SUB + s * 2 * NSUB, 2 * NSUB)
        pltpu.sync_copy(idx_hbm.at[pl.ds(irow, 2 * NSUB)], idx_v)

    def fire(g, p):
        for j in range(NSUB):
            pltpu.async_copy(
                table_hbm.at[idx_v.at[(g % 2) * NSUB + j]],
                gbufs[p].at[pl.ds(j * SUB, SUB)],
                gsem,
            )

    def wait_gathers(p):
        for j in range(NSUB):
            pltpu.make_async_copy(
                table_hbm.at[idx_v.at[j]],
                gbufs[p].at[pl.ds(j * SUB, SUB)],
                gsem,
            ).wait()

    def obase(g):
        return pl.multiple_of(base + g * C, C)

    def wait_out(p):
        pltpu.make_async_copy(
            obufs[p], out_hbm.at[pl.ds(0, C)], osem).wait()

    def pool(p):
        gbuf, obuf = gbufs[p], obufs[p]

        def row(n, carry):
            r = n * D_
            for q in range(QE):
                sl = pl.ds(q * LANES, LANES)
                acc = gbuf[r, sl]
                for k in range(1, D_):
                    acc = acc + gbuf[r + k, sl]
                obuf[n, sl] = acc
            return carry

        lax.fori_loop(0, C, row, 0, unroll=2)

    def out_copy(g, p):
        pltpu.async_copy(obufs[p], out_hbm.at[pl.ds(obase(g), C)], osem)

    stage(0)
    fire(0, 0)

    def super_body(go, carry):
        g = 2 * go
        fire(g + 1, 1)
        wait_gathers(0)

        @pl.when(go >= 1)
        def _():
            wait_out(0)

        pool(0)
        out_copy(g, 0)

        wait_gathers(1)
        stage(go + 1)
        fire(g + 2, 0)

        @pl.when(go >= 1)
        def _():
            wait_out(1)

        pool(1)
        out_copy(g + 1, 1)
        return carry

    lax.fori_loop(0, NSUPER - 1, super_body, 0)

    fire(NCHUNK - 1, 1)
    wait_gathers(0)
    wait_out(0)
    pool(0)
    out_copy(NCHUNK - 2, 0)
    wait_gathers(1)
    wait_out(1)
    pool(1)
    out_copy(NCHUNK - 1, 1)
    wait_out(0)
    wait_out(1)


def kernel(input, table):
    b, s, d = input.shape
    flat_idx = input.reshape(b * s * d // SUB, SUB)
    out = _emb_pool(flat_idx, table)
    return out[:, :EMBED].reshape(b, s, EMBED)
